# baseline (speedup 1.0000x reference)
import functools

import jax
import jax.numpy as jnp
from jax import lax
from jax.experimental import pallas as pl
from jax.experimental.pallas import tpu as pltpu
from jax.experimental.pallas import tpu_sc as plsc

E = 8
TOPK = 2
H = 1024
I = 2048
S = 2048
P = S * TOPK

BM = 256
BN = 512
NB = P // BM
NN = I // BN
MAX_STEPS = NB + E - 1

PA = 128
PB = P // PA


def _tri(n, dtype=jnp.float32):
    r = lax.broadcasted_iota(jnp.int32, (n, n), 0)
    c = lax.broadcasted_iota(jnp.int32, (n, n), 1)
    return (r < c).astype(dtype)


def _router_body(x_ref, g_ref, rw_ref, rank_ref, st_ref):
    x = x_ref[...]
    g = g_ref[...]
    logits = lax.dot_general(x, g, (((1,), (1,)), ((), ())),
                             preferred_element_type=jnp.float32)
    m = jnp.max(logits, axis=1, keepdims=True)
    p = jnp.exp(logits - m)
    probs = p / jnp.sum(p, axis=1, keepdims=True)

    cols = lax.broadcasted_iota(jnp.int32, (S, E), 1)
    v1 = jnp.max(probs, axis=1, keepdims=True)
    i1 = jnp.min(jnp.where(probs == v1, cols, E), axis=1, keepdims=True)
    probs2 = jnp.where(cols == i1, -jnp.inf, probs)
    v2 = jnp.max(probs2, axis=1, keepdims=True)
    i2 = jnp.min(jnp.where(probs2 == v2, cols, E), axis=1, keepdims=True)

    denom = v1 + v2
    rw_ref[:, 0:1] = v1 / denom
    rw_ref[:, 1:2] = v2 / denom

    eidx = lax.broadcasted_iota(jnp.int32, (S, E), 1)
    oh1 = (eidx == i1).astype(jnp.float32)
    oh2 = (eidx == i2).astype(jnp.float32)
    ohsum = oh1 + oh2
    bpre = lax.dot_general(_tri(S), ohsum, (((0,), (0,)), ((), ())),
                           preferred_element_type=jnp.float32)
    counts = jnp.sum(ohsum, axis=0, keepdims=True)
    offs = lax.dot_general(counts, _tri(E), (((1,), (0,)), ((), ())),
                           precision=lax.Precision.HIGHEST,
                           preferred_element_type=jnp.float32)

    basef = offs + bpre
    rank_ref[:, 0:1] = jnp.sum(oh1 * basef, axis=1,
                               keepdims=True).astype(jnp.int32)
    rank_ref[:, 1:2] = jnp.sum(oh2 * basef, axis=1,
                               keepdims=True).astype(jnp.int32)

    offs_v = offs[0]
    ends_v = offs_v + counts[0]
    mlo = lax.broadcasted_iota(jnp.int32, (NB, E), 0).astype(jnp.float32) * BM
    valid = (offs_v[None, :] < mlo + BM) & (ends_v[None, :] > mlo)
    validf = valid.astype(jnp.float32)
    rowcnt = jnp.sum(validf, axis=1, keepdims=True)
    rowpre = lax.dot_general(_tri(NB), rowcnt, (((0,), (0,)), ((), ())),
                             precision=lax.Precision.HIGHEST,
                           preferred_element_type=jnp.float32)
    colpre = lax.dot_general(validf, _tri(E), (((1,), (0,)), ((), ())),
                             precision=lax.Precision.HIGHEST,
                           preferred_element_type=jnp.float32)
    pos = rowpre + colpre
    eidx2 = lax.broadcasted_iota(jnp.int32, (NB, E), 1).astype(jnp.float32)
    midx2 = lax.broadcasted_iota(jnp.int32, (NB, E), 0).astype(jnp.float32)
    e_last = jnp.max(jnp.where(valid, eidx2, -1.0))

    prev_sm = jnp.float32(-1.0)
    sm_list = []
    for s in range(MAX_STEPS):
        sel = validf * (pos == s).astype(jnp.float32)
        live = jnp.sum(sel) > 0.5
        se_s = jnp.where(live, jnp.sum(eidx2 * sel), e_last)
        sm_s = jnp.where(live, jnp.sum(midx2 * sel), float(NB - 1))
        lo_s = jnp.where(live, jnp.sum(offs_v[None, :] * sel), 0.0)
        hi_s = jnp.where(live, jnp.sum(ends_v[None, :] * sel), 0.0)
        sf_s = jnp.where(live & (sm_s != prev_sm), 1.0, 0.0)
        prev_sm = sm_s
        sm_list.append(sm_s)
        st_ref[0, s] = se_s.astype(jnp.int32)
        st_ref[1, s] = sm_s.astype(jnp.int32)
        st_ref[2, s] = sf_s.astype(jnp.int32)
        st_ref[4, s] = lo_s.astype(jnp.int32)
        st_ref[5, s] = hi_s.astype(jnp.int32)
    for s in range(MAX_STEPS):
        if s == MAX_STEPS - 1:
            sl_s = jnp.float32(1.0)
        else:
            sl_s = jnp.where(sm_list[s + 1] != sm_list[s], 1.0, 0.0)
        st_ref[3, s] = sl_s.astype(jnp.int32)


def _router(x, gate_weight):
    return pl.pallas_call(
        _router_body,
        out_shape=(
            jax.ShapeDtypeStruct((S, TOPK), jnp.float32),
            jax.ShapeDtypeStruct((S, TOPK), jnp.int32),
            jax.ShapeDtypeStruct((6, MAX_STEPS), jnp.int32),
        ),
        out_specs=(
            pl.BlockSpec((S, TOPK), lambda: (0, 0)),
            pl.BlockSpec((S, TOPK), lambda: (0, 0)),
            pl.BlockSpec(memory_space=pltpu.SMEM),
        ),
    )(x, gate_weight)


def _gemm_body(se_ref, sm_ref, sf_ref, sl_ref, slo_ref, shi_ref,
               x_ref, w1_ref, w3_ref, w2_ref, rw_ref, out_ref, acc_ref):
    n = pl.program_id(0)
    i = pl.program_id(1)
    first = sf_ref[i]
    last = sl_ref[i]
    lo = slo_ref[i]
    hi = shi_ref[i]
    row0 = sm_ref[i] * BM

    x = x_ref[...]
    h1 = jnp.dot(x, w1_ref[0], preferred_element_type=jnp.float32)
    h3 = jnp.dot(x, w3_ref[0], preferred_element_type=jnp.float32)
    rows = row0 + lax.broadcasted_iota(jnp.int32, (BM, 1), 0)
    mask = (rows >= lo) & (rows < hi)
    prod = (h1 * jax.nn.sigmoid(h1)) * h3
    c = jnp.dot(prod, w2_ref[0], preferred_element_type=jnp.float32)
    c = c * jnp.where(mask, rw_ref[...], 0.0)

    sl = pl.ds(row0, BM)
    final_write = (n == NN - 1) & (last == 1)

    @pl.when((n == 0) & (first == 1))
    def _():
        acc_ref[sl, :] = c

    @pl.when(jnp.logical_not((n == 0) & (first == 1))
             & jnp.logical_not(final_write))
    def _():
        acc_ref[sl, :] += c

    @pl.when(final_write)
    def _():
        out_ref[...] = acc_ref[sl, :] + c


def _grouped_gemm(x_sorted, rw_sorted, w1s, w3s, w2s, se, sm, sf, sl, slo, shi):
    grid_spec = pltpu.PrefetchScalarGridSpec(
        num_scalar_prefetch=6,
        grid=(NN, MAX_STEPS),
        in_specs=[
            pl.BlockSpec((BM, H),
                         lambda n, i, se, sm, sf, sl, lo, hi: (sm[i], 0)),
            pl.BlockSpec((1, H, BN),
                         lambda n, i, se, sm, sf, sl, lo, hi: (se[i], 0, n)),
            pl.BlockSpec((1, H, BN),
                         lambda n, i, se, sm, sf, sl, lo, hi: (se[i], 0, n)),
            pl.BlockSpec((1, BN, H),
                         lambda n, i, se, sm, sf, sl, lo, hi: (se[i], n, 0)),
            pl.BlockSpec((BM, 1),
                         lambda n, i, se, sm, sf, sl, lo, hi: (sm[i], 0)),
        ],
        out_specs=pl.BlockSpec(
            (BM, H),
            lambda n, i, se, sm, sf, sl, lo, hi: (
                jnp.where(n == NN - 1, sm[i], 0), 0)),
        scratch_shapes=[pltpu.VMEM((P, H), jnp.float32)],
    )
    return pl.pallas_call(
        _gemm_body,
        grid_spec=grid_spec,
        out_shape=jax.ShapeDtypeStruct((P, H), jnp.float32),
    )(se, sm, sf, sl, slo, shi, x_sorted, w1s, w3s, w2s, rw_sorted)


NW = 32
RPW = P // NW
TPW = S // NW
CH = 16

_SC_MESH = dict(core_axis_name="c", subcore_axis_name="s")


def _sc_gather_body(x_hbm, rank_hbm, rw_hbm, xs_hbm, rws_hbm,
                    rank_v, rw_v, src_v, rws_v, buf, sem0, sem1):
    wid = lax.axis_index("s") * 2 + lax.axis_index("c")
    base = wid * RPW

    pltpu.sync_copy(rank_hbm, rank_v)
    pltpu.sync_copy(rw_hbm, rw_v)

    def scat(i, carry):
        vr = rank_v[pl.ds(i * 16, 16)]
        tok = lax.shift_right_logical(lax.iota(jnp.int32, 16) + i * 16, 1)
        plsc.store_scatter(src_v, [vr], tok)
        plsc.store_scatter(rws_v, [vr], rw_v[pl.ds(i * 16, 16)])
        return carry

    lax.fori_loop(0, P // 16, scat, 0)

    pltpu.sync_copy(rws_v.at[pl.ds(base, RPW)], rws_hbm.at[pl.ds(base, RPW)])

    sems = (sem0, sem1)

    def fire(ci, slot):
        idxv = src_v[pl.ds(base + ci * CH, CH)]
        return pltpu.async_copy(x_hbm.at[idxv], buf.at[slot], sems[slot])

    nch = RPW // CH
    descs = {0: fire(0, 0)}
    for ci in range(nch):
        if ci + 1 < nch:
            descs[ci + 1] = fire(ci + 1, (ci + 1) % 2)
        descs[ci].wait()
        pltpu.sync_copy(buf.at[ci % 2], xs_hbm.at[pl.ds(base + ci * CH, CH)])


def _sc_gather(x, rank, rw_flat):
    k = pl.kernel(
        _sc_gather_body,
        out_type=(
            jax.ShapeDtypeStruct((P, H), jnp.float32),
            jax.ShapeDtypeStruct((P,), jnp.float32),
        ),
        mesh=plsc.VectorSubcoreMesh(**_SC_MESH),
        compiler_params=pltpu.CompilerParams(needs_layout_passes=False),
        scratch_types=(
            pltpu.VMEM((P,), jnp.int32),
            pltpu.VMEM((P,), jnp.float32),
            pltpu.VMEM((P,), jnp.int32),
            pltpu.VMEM((P,), jnp.float32),
            pltpu.VMEM((2, CH, H), jnp.float32),
            pltpu.SemaphoreType.DMA,
            pltpu.SemaphoreType.DMA,
        ),
    )
    return k(x, rank, rw_flat)


def _sc_combine_body(os_hbm, rank_hbm, fin_hbm, rank_v, pbuf, obuf, sem0, sem1):
    wid = lax.axis_index("s") * 2 + lax.axis_index("c")
    pbase = wid * RPW
    tbase = wid * TPW

    pltpu.sync_copy(rank_hbm.at[pl.ds(pbase, RPW)], rank_v)
    sems = (sem0, sem1)

    def fire(ci, slot):
        idxv = rank_v[pl.ds(ci * CH, CH)]
        return pltpu.async_copy(os_hbm.at[idxv], pbuf.at[slot], sems[slot])

    nch = RPW // CH
    descs = {0: fire(0, 0)}
    for ci in range(nch):
        if ci + 1 < nch:
            descs[ci + 1] = fire(ci + 1, (ci + 1) % 2)
        descs[ci].wait()
        slot = ci % 2

        def addk(kk, carry):
            sl = pl.ds(kk * 16, 16)
            for j in range(CH // 2):
                obuf[j, sl] = pbuf[slot, 2 * j, sl] + pbuf[slot, 2 * j + 1, sl]
            return carry

        lax.fori_loop(0, H // 16, addk, 0)
        pltpu.sync_copy(obuf, fin_hbm.at[pl.ds(tbase + ci * (CH // 2), CH // 2)])


def _sc_combine(out_sorted, rank):
    k = pl.kernel(
        _sc_combine_body,
        out_type=jax.ShapeDtypeStruct((S, H), jnp.float32),
        mesh=plsc.VectorSubcoreMesh(**_SC_MESH),
        compiler_params=pltpu.CompilerParams(needs_layout_passes=False),
        scratch_types=(
            pltpu.VMEM((RPW,), jnp.int32),
            pltpu.VMEM((2, CH, H), jnp.float32),
            pltpu.VMEM((CH // 2, H), jnp.float32),
            pltpu.SemaphoreType.DMA,
            pltpu.SemaphoreType.DMA,
        ),
    )
    return k(out_sorted, rank)


def kernel(hidden_states, gate_weight, w1s, w2s, w3s):
    b, s, h = hidden_states.shape
    x = hidden_states.reshape(S, H)

    rw, rank2d, st = _router(x, gate_weight)
    rank = rank2d.reshape(P)
    se, sm, sf, sl, slo, shi = st[0], st[1], st[2], st[3], st[4], st[5]

    x_sorted, rw_sorted = _sc_gather(x, rank, rw.reshape(P))

    out_sorted = _grouped_gemm(x_sorted, rw_sorted[:, None], w1s, w3s, w2s,
                               se, sm, sf, sl, slo, shi)

    final = _sc_combine(out_sorted, rank)
    return final.reshape(b, s, h)


# speedup vs baseline: 1.2052x; 1.2052x over previous
import functools

import jax
import jax.numpy as jnp
from jax import lax
from jax.experimental import pallas as pl
from jax.experimental.pallas import tpu as pltpu
from jax.experimental.pallas import tpu_sc as plsc

E = 8
TOPK = 2
H = 1024
I = 2048
S = 2048
P = S * TOPK

BM = 512
BN = 1024
NB = P // BM
NN = I // BN
MAX_STEPS = NB + E - 1

PA = 128
PB = P // PA


def _tri(n, dtype=jnp.float32):
    r = lax.broadcasted_iota(jnp.int32, (n, n), 0)
    c = lax.broadcasted_iota(jnp.int32, (n, n), 1)
    return (r < c).astype(dtype)


def _router_body(x_ref, g_ref, rw_ref, rank_ref, st_ref):
    x = x_ref[...]
    g = g_ref[...]
    logits = lax.dot_general(x, g, (((1,), (1,)), ((), ())),
                             preferred_element_type=jnp.float32)
    m = jnp.max(logits, axis=1, keepdims=True)
    p = jnp.exp(logits - m)
    probs = p / jnp.sum(p, axis=1, keepdims=True)

    cols = lax.broadcasted_iota(jnp.int32, (S, E), 1)
    v1 = jnp.max(probs, axis=1, keepdims=True)
    i1 = jnp.min(jnp.where(probs == v1, cols, E), axis=1, keepdims=True)
    probs2 = jnp.where(cols == i1, -jnp.inf, probs)
    v2 = jnp.max(probs2, axis=1, keepdims=True)
    i2 = jnp.min(jnp.where(probs2 == v2, cols, E), axis=1, keepdims=True)

    denom = v1 + v2
    rw_ref[:, 0:1] = v1 / denom
    rw_ref[:, 1:2] = v2 / denom

    eidx = lax.broadcasted_iota(jnp.int32, (S, E), 1)
    oh1 = (eidx == i1).astype(jnp.float32)
    oh2 = (eidx == i2).astype(jnp.float32)
    ohsum = oh1 + oh2
    bpre = lax.dot_general(_tri(S), ohsum, (((0,), (0,)), ((), ())),
                           preferred_element_type=jnp.float32)
    counts = jnp.sum(ohsum, axis=0, keepdims=True)
    offs = lax.dot_general(counts, _tri(E), (((1,), (0,)), ((), ())),
                           precision=lax.Precision.HIGHEST,
                           preferred_element_type=jnp.float32)

    basef = offs + bpre
    rank_ref[:, 0:1] = jnp.sum(oh1 * basef, axis=1,
                               keepdims=True).astype(jnp.int32)
    rank_ref[:, 1:2] = jnp.sum(oh2 * basef, axis=1,
                               keepdims=True).astype(jnp.int32)

    offs_v = offs[0]
    ends_v = offs_v + counts[0]
    mlo = lax.broadcasted_iota(jnp.int32, (NB, E), 0).astype(jnp.float32) * BM
    valid = (offs_v[None, :] < mlo + BM) & (ends_v[None, :] > mlo)
    validf = valid.astype(jnp.float32)
    rowcnt = jnp.sum(validf, axis=1, keepdims=True)
    rowpre = lax.dot_general(_tri(NB), rowcnt, (((0,), (0,)), ((), ())),
                             precision=lax.Precision.HIGHEST,
                           preferred_element_type=jnp.float32)
    colpre = lax.dot_general(validf, _tri(E), (((1,), (0,)), ((), ())),
                             precision=lax.Precision.HIGHEST,
                           preferred_element_type=jnp.float32)
    pos = rowpre + colpre
    eidx2 = lax.broadcasted_iota(jnp.int32, (NB, E), 1).astype(jnp.float32)
    midx2 = lax.broadcasted_iota(jnp.int32, (NB, E), 0).astype(jnp.float32)
    e_last = jnp.max(jnp.where(valid, eidx2, -1.0))

    prev_sm = jnp.float32(-1.0)
    sm_list = []
    for s in range(MAX_STEPS):
        sel = validf * (pos == s).astype(jnp.float32)
        live = jnp.sum(sel) > 0.5
        se_s = jnp.where(live, jnp.sum(eidx2 * sel), e_last)
        sm_s = jnp.where(live, jnp.sum(midx2 * sel), float(NB - 1))
        lo_s = jnp.where(live, jnp.sum(offs_v[None, :] * sel), 0.0)
        hi_s = jnp.where(live, jnp.sum(ends_v[None, :] * sel), 0.0)
        sf_s = jnp.where(live & (sm_s != prev_sm), 1.0, 0.0)
        prev_sm = sm_s
        sm_list.append(sm_s)
        st_ref[0, s] = se_s.astype(jnp.int32)
        st_ref[1, s] = sm_s.astype(jnp.int32)
        st_ref[2, s] = sf_s.astype(jnp.int32)
        st_ref[4, s] = lo_s.astype(jnp.int32)
        st_ref[5, s] = hi_s.astype(jnp.int32)
    for s in range(MAX_STEPS):
        if s == MAX_STEPS - 1:
            sl_s = jnp.float32(1.0)
        else:
            sl_s = jnp.where(sm_list[s + 1] != sm_list[s], 1.0, 0.0)
        st_ref[3, s] = sl_s.astype(jnp.int32)


def _router(x, gate_weight):
    return pl.pallas_call(
        _router_body,
        out_shape=(
            jax.ShapeDtypeStruct((S, TOPK), jnp.float32),
            jax.ShapeDtypeStruct((S, TOPK), jnp.int32),
            jax.ShapeDtypeStruct((6, MAX_STEPS), jnp.int32),
        ),
        out_specs=(
            pl.BlockSpec((S, TOPK), lambda: (0, 0)),
            pl.BlockSpec((S, TOPK), lambda: (0, 0)),
            pl.BlockSpec(memory_space=pltpu.SMEM),
        ),
    )(x, gate_weight)


def _gemm_body(se_ref, sm_ref, sf_ref, sl_ref, slo_ref, shi_ref,
               x_ref, w1_ref, w3_ref, w2_ref, rw_ref, out_ref, acc_ref):
    n = pl.program_id(0)
    i = pl.program_id(1)
    first = sf_ref[i]
    last = sl_ref[i]
    lo = slo_ref[i]
    hi = shi_ref[i]
    row0 = sm_ref[i] * BM

    x = x_ref[...]
    h1 = jnp.dot(x, w1_ref[0], preferred_element_type=jnp.float32)
    h3 = jnp.dot(x, w3_ref[0], preferred_element_type=jnp.float32)
    rows = row0 + lax.broadcasted_iota(jnp.int32, (BM, 1), 0)
    mask = (rows >= lo) & (rows < hi)
    prod = (h1 * jax.nn.sigmoid(h1)) * h3
    c = jnp.dot(prod, w2_ref[0], preferred_element_type=jnp.float32)
    c = c * jnp.where(mask, rw_ref[...], 0.0)

    sl = pl.ds(row0, BM)
    final_write = (n == NN - 1) & (last == 1)

    @pl.when((n == 0) & (first == 1))
    def _():
        acc_ref[sl, :] = c

    @pl.when(jnp.logical_not((n == 0) & (first == 1))
             & jnp.logical_not(final_write))
    def _():
        acc_ref[sl, :] += c

    @pl.when(final_write)
    def _():
        out_ref[...] = acc_ref[sl, :] + c


def _grouped_gemm(x_sorted, rw_sorted, w1s, w3s, w2s, se, sm, sf, sl, slo, shi):
    grid_spec = pltpu.PrefetchScalarGridSpec(
        num_scalar_prefetch=6,
        grid=(NN, MAX_STEPS),
        in_specs=[
            pl.BlockSpec((BM, H),
                         lambda n, i, se, sm, sf, sl, lo, hi: (sm[i], 0)),
            pl.BlockSpec((1, H, BN),
                         lambda n, i, se, sm, sf, sl, lo, hi: (se[i], 0, n)),
            pl.BlockSpec((1, H, BN),
                         lambda n, i, se, sm, sf, sl, lo, hi: (se[i], 0, n)),
            pl.BlockSpec((1, BN, H),
                         lambda n, i, se, sm, sf, sl, lo, hi: (se[i], n, 0)),
            pl.BlockSpec((BM, 1),
                         lambda n, i, se, sm, sf, sl, lo, hi: (sm[i], 0)),
        ],
        out_specs=pl.BlockSpec(
            (BM, H),
            lambda n, i, se, sm, sf, sl, lo, hi: (
                jnp.where(n == NN - 1, sm[i], 0), 0)),
        scratch_shapes=[pltpu.VMEM((P, H), jnp.float32)],
    )
    return pl.pallas_call(
        _gemm_body,
        grid_spec=grid_spec,
        out_shape=jax.ShapeDtypeStruct((P, H), jnp.float32),
    )(se, sm, sf, sl, slo, shi, x_sorted, w1s, w3s, w2s, rw_sorted)


NW = 32
RPW = P // NW
TPW = S // NW
CH = 16

_SC_MESH = dict(core_axis_name="c", subcore_axis_name="s")


def _sc_gather_body(x_hbm, rank_hbm, rw_hbm, xs_hbm, rws_hbm,
                    rank_v, rw_v, src_v, rws_v, buf, sem0, sem1):
    wid = lax.axis_index("s") * 2 + lax.axis_index("c")
    base = wid * RPW

    pltpu.sync_copy(rank_hbm, rank_v)
    pltpu.sync_copy(rw_hbm, rw_v)

    def scat(i, carry):
        vr = rank_v[pl.ds(i * 16, 16)]
        tok = lax.shift_right_logical(lax.iota(jnp.int32, 16) + i * 16, 1)
        plsc.store_scatter(src_v, [vr], tok)
        plsc.store_scatter(rws_v, [vr], rw_v[pl.ds(i * 16, 16)])
        return carry

    lax.fori_loop(0, P // 16, scat, 0)

    pltpu.sync_copy(rws_v.at[pl.ds(base, RPW)], rws_hbm.at[pl.ds(base, RPW)])

    sems = (sem0, sem1)

    def fire(ci, slot):
        idxv = src_v[pl.ds(base + ci * CH, CH)]
        return pltpu.async_copy(x_hbm.at[idxv], buf.at[slot], sems[slot])

    nch = RPW // CH
    descs = {0: fire(0, 0)}
    for ci in range(nch):
        if ci + 1 < nch:
            descs[ci + 1] = fire(ci + 1, (ci + 1) % 2)
        descs[ci].wait()
        pltpu.sync_copy(buf.at[ci % 2], xs_hbm.at[pl.ds(base + ci * CH, CH)])


def _sc_gather(x, rank, rw_flat):
    k = pl.kernel(
        _sc_gather_body,
        out_type=(
            jax.ShapeDtypeStruct((P, H), jnp.float32),
            jax.ShapeDtypeStruct((P,), jnp.float32),
        ),
        mesh=plsc.VectorSubcoreMesh(**_SC_MESH),
        compiler_params=pltpu.CompilerParams(needs_layout_passes=False),
        scratch_types=(
            pltpu.VMEM((P,), jnp.int32),
            pltpu.VMEM((P,), jnp.float32),
            pltpu.VMEM((P,), jnp.int32),
            pltpu.VMEM((P,), jnp.float32),
            pltpu.VMEM((2, CH, H), jnp.float32),
            pltpu.SemaphoreType.DMA,
            pltpu.SemaphoreType.DMA,
        ),
    )
    return k(x, rank, rw_flat)


def _sc_combine_body(os_hbm, rank_hbm, fin_hbm, rank_v, pbuf, obuf, sem0, sem1):
    wid = lax.axis_index("s") * 2 + lax.axis_index("c")
    pbase = wid * RPW
    tbase = wid * TPW

    pltpu.sync_copy(rank_hbm.at[pl.ds(pbase, RPW)], rank_v)
    sems = (sem0, sem1)

    def fire(ci, slot):
        idxv = rank_v[pl.ds(ci * CH, CH)]
        return pltpu.async_copy(os_hbm.at[idxv], pbuf.at[slot], sems[slot])

    nch = RPW // CH
    descs = {0: fire(0, 0)}
    for ci in range(nch):
        if ci + 1 < nch:
            descs[ci + 1] = fire(ci + 1, (ci + 1) % 2)
        descs[ci].wait()
        slot = ci % 2

        def addk(kk, carry):
            sl = pl.ds(kk * 16, 16)
            for j in range(CH // 2):
                obuf[j, sl] = pbuf[slot, 2 * j, sl] + pbuf[slot, 2 * j + 1, sl]
            return carry

        lax.fori_loop(0, H // 16, addk, 0)
        pltpu.sync_copy(obuf, fin_hbm.at[pl.ds(tbase + ci * (CH // 2), CH // 2)])


def _sc_combine(out_sorted, rank):
    k = pl.kernel(
        _sc_combine_body,
        out_type=jax.ShapeDtypeStruct((S, H), jnp.float32),
        mesh=plsc.VectorSubcoreMesh(**_SC_MESH),
        compiler_params=pltpu.CompilerParams(needs_layout_passes=False),
        scratch_types=(
            pltpu.VMEM((RPW,), jnp.int32),
            pltpu.VMEM((2, CH, H), jnp.float32),
            pltpu.VMEM((CH // 2, H), jnp.float32),
            pltpu.SemaphoreType.DMA,
            pltpu.SemaphoreType.DMA,
        ),
    )
    return k(out_sorted, rank)


def kernel(hidden_states, gate_weight, w1s, w2s, w3s):
    b, s, h = hidden_states.shape
    x = hidden_states.reshape(S, H)

    rw, rank2d, st = _router(x, gate_weight)
    rank = rank2d.reshape(P)
    se, sm, sf, sl, slo, shi = st[0], st[1], st[2], st[3], st[4], st[5]

    x_sorted, rw_sorted = _sc_gather(x, rank, rw.reshape(P))

    out_sorted = _grouped_gemm(x_sorted, rw_sorted[:, None], w1s, w3s, w2s,
                               se, sm, sf, sl, slo, shi)

    final = _sc_combine(out_sorted, rank)
    return final.reshape(b, s, h)
